# baseline (speedup 1.0000x reference)
import functools

import jax
import jax.numpy as jnp
from jax import lax
from jax.experimental import pallas as pl
from jax.experimental.pallas import tpu as pltpu
from jax.experimental.pallas import tpu_sc as plsc

E, B, D, P, K = 16, 4096, 1024, 4, 2

BG = 1024
BT = 256
SCB = 1024
NC, NS, L = 2, 16, 16
NW = NC * NS
TK = 2


def _gate_body(x_ref, w_ref, b_ref, pw_ref, sp_ref, spb_ref,
               soft_ref, hard_ref):
    x = x_ref[...]
    w = w_ref[...]
    logits = lax.dot_general(
        x, w, (((1,), (1,)), ((), ())),
        preferred_element_type=jnp.float32) + b_ref[...]

    col = lax.broadcasted_iota(jnp.int32, (BG, E), 1)
    rank = jnp.zeros((BG, E), dtype=jnp.int32)
    for j in range(E):
        lj = logits[:, j:j + 1]
        gt = (lj > logits).astype(jnp.int32)
        eq = ((lj == logits) & (j < col)).astype(jnp.int32)
        rank = rank + gt + eq
    keep = rank < K

    scattered = jnp.where(keep, logits, 0.0)
    masked = jnp.where(scattered == 0.0, -jnp.inf, scattered)
    m = jnp.max(masked, axis=1, keepdims=True)
    ex = jnp.exp(masked - m)
    s = ex / jnp.sum(ex, axis=1, keepdims=True)

    perm = (pw_ref[0] + pw_ref[1] + pw_ref[2] + pw_ref[3]) * 0.25
    sp = lax.dot_general(
        s, perm, (((1,), (0,)), ((), ())),
        preferred_element_type=jnp.float32)

    sp_ref[...] = sp
    spb_ref[...] = jnp.broadcast_to(sp[:, :, None], (BG, E, L))
    soft_ref[...] = jnp.sum(sp, axis=0, keepdims=True)[None]
    hard_ref[...] = jnp.sum((sp >= 1e-5).astype(jnp.float32), axis=0,
                            keepdims=True)[None]


def _combine_body(sp_ref, h_ref, y_ref):
    sp = sp_ref[...]
    acc = h_ref[0] * sp[:, 0:1]
    for e in range(1, E):
        acc = acc + h_ref[e] * sp[:, e:e + 1]
    y_ref[...] = acc


def _reduce_body(soft_ref, hard_ref, savg_ref, havg_ref):
    savg_ref[...] = jnp.sum(soft_ref[...], axis=0, keepdims=True) * (1.0 / B)
    havg_ref[...] = jnp.sum(hard_ref[...], axis=0, keepdims=True) * (1.0 / B)


def _sc_combine_body(h_hbm, spb_hbm, y_hbm, hbuf, spbuf, ybuf, sem):
    wid = lax.axis_index("s") * NC + lax.axis_index("c")
    per_w = SCB // NW
    base = wid * per_w

    def chunk(i, carry):
        b = base + i * TK
        cps = [pltpu.async_copy(h_hbm.at[e, pl.ds(b, TK), :], hbuf.at[e], sem)
               for e in range(E)]
        sp_cp = pltpu.async_copy(spb_hbm.at[pl.ds(b, TK)], spbuf, sem)
        for cp in cps:
            cp.wait()
        sp_cp.wait()
        for t in range(TK):
            spv = [spbuf[t, e] for e in range(E)]

            def col(c, carry2):
                off = c * L
                acc = spv[0] * hbuf[0, t, pl.ds(off, L)]
                for e in range(1, E):
                    acc = acc + spv[e] * hbuf[e, t, pl.ds(off, L)]
                ybuf[t, pl.ds(off, L)] = acc
                return carry2

            lax.fori_loop(0, D // L, col, 0)
        pltpu.sync_copy(ybuf, y_hbm.at[pl.ds(b, TK), :])
        return carry

    lax.fori_loop(0, per_w // TK, chunk, 0)


_sc_combine = functools.partial(
    pl.kernel,
    out_type=jax.ShapeDtypeStruct((SCB, D), jnp.float32),
    mesh=plsc.VectorSubcoreMesh(core_axis_name="c", subcore_axis_name="s"),
    scratch_types=[
        pltpu.VMEM((E, TK, D), jnp.float32),
        pltpu.VMEM((TK, E, L), jnp.float32),
        pltpu.VMEM((TK, D), jnp.float32),
        pltpu.SemaphoreType.DMA,
    ],
)(_sc_combine_body)


@functools.partial(jax.jit)
def kernel(h, x, permutation_weights, expert_weights, bias):
    bias2d = bias.reshape(1, E)
    ntg = B // BG
    sp, spb, soft, hard = pl.pallas_call(
        _gate_body,
        grid=(ntg,),
        in_specs=[
            pl.BlockSpec((BG, D), lambda i: (i, 0)),
            pl.BlockSpec((E, D), lambda i: (0, 0)),
            pl.BlockSpec((1, E), lambda i: (0, 0)),
            pl.BlockSpec((P, E, E), lambda i: (0, 0, 0)),
        ],
        out_specs=[
            pl.BlockSpec((BG, E), lambda i: (i, 0)),
            pl.BlockSpec((BG, E, L), lambda i: (i, 0, 0)),
            pl.BlockSpec((1, 1, E), lambda i: (i, 0, 0)),
            pl.BlockSpec((1, 1, E), lambda i: (i, 0, 0)),
        ],
        out_shape=[
            jax.ShapeDtypeStruct((B, E), jnp.float32),
            jax.ShapeDtypeStruct((B, E, L), jnp.float32),
            jax.ShapeDtypeStruct((ntg, 1, E), jnp.float32),
            jax.ShapeDtypeStruct((ntg, 1, E), jnp.float32),
        ],
        compiler_params=pltpu.CompilerParams(
            dimension_semantics=("arbitrary",),
        ),
    )(x, expert_weights, bias2d, permutation_weights)

    y_sc = _sc_combine(h, spb)

    off = SCB // BT
    ntc = (B - SCB) // BT
    y_tc = pl.pallas_call(
        _combine_body,
        grid=(ntc,),
        in_specs=[
            pl.BlockSpec((BT, E), lambda i: (i + off, 0)),
            pl.BlockSpec((E, BT, D), lambda i: (0, i + off, 0)),
        ],
        out_specs=pl.BlockSpec((BT, D), lambda i: (i, 0)),
        out_shape=jax.ShapeDtypeStruct((B - SCB, D), jnp.float32),
        compiler_params=pltpu.CompilerParams(
            dimension_semantics=("arbitrary",),
        ),
    )(sp, h)

    savg, havg = pl.pallas_call(
        _reduce_body,
        out_shape=[
            jax.ShapeDtypeStruct((1, E), jnp.float32),
            jax.ShapeDtypeStruct((1, E), jnp.float32),
        ],
    )(soft.reshape(ntg, E), hard.reshape(ntg, E))

    y = jnp.concatenate([y_sc, y_tc], axis=0)
    return (y, savg.reshape(E, 1), havg.reshape(E, 1))


# speedup vs baseline: 1.5939x; 1.5939x over previous
import functools

import jax
import jax.numpy as jnp
from jax.experimental import pallas as pl
from jax.experimental.pallas import tpu as pltpu

E, B, D, P, K = 16, 4096, 1024, 4, 2
BT = 256


def _body(x_ref, w_ref, b_ref, pw_ref, h_ref, y_ref, soft_ref, hard_ref):
    i = pl.program_id(0)
    nsteps = pl.num_programs(0)

    x = x_ref[...]
    w = w_ref[...]
    logits = jax.lax.dot_general(
        x, w, (((1,), (1,)), ((), ())),
        preferred_element_type=jnp.float32) + b_ref[...]

    col = jax.lax.broadcasted_iota(jnp.int32, (BT, E), 1)
    rank = jnp.zeros((BT, E), dtype=jnp.int32)
    for j in range(E):
        lj = logits[:, j:j + 1]
        gt = (lj > logits).astype(jnp.int32)
        eq = ((lj == logits) & (j < col)).astype(jnp.int32)
        rank = rank + gt + eq
    keep = rank < K

    scattered = jnp.where(keep, logits, 0.0)
    masked = jnp.where(scattered == 0.0, -jnp.inf, scattered)
    m = jnp.max(masked, axis=1, keepdims=True)
    ex = jnp.exp(masked - m)
    s = ex / jnp.sum(ex, axis=1, keepdims=True)

    perm = (pw_ref[0] + pw_ref[1] + pw_ref[2] + pw_ref[3]) * 0.25
    sp = jax.lax.dot_general(
        s, perm, (((1,), (0,)), ((), ())),
        preferred_element_type=jnp.float32)

    acc = h_ref[0] * sp[:, 0:1]
    for e in range(1, E):
        acc = acc + h_ref[e] * sp[:, e:e + 1]
    y_ref[...] = acc

    @pl.when(i == 0)
    def _init():
        soft_ref[...] = jnp.zeros_like(soft_ref)
        hard_ref[...] = jnp.zeros_like(hard_ref)

    soft_ref[...] += jnp.sum(sp, axis=0, keepdims=True)
    hard_ref[...] += jnp.sum((sp >= 1e-5).astype(jnp.float32), axis=0,
                             keepdims=True)

    @pl.when(i == nsteps - 1)
    def _finalize():
        soft_ref[...] = soft_ref[...] * (1.0 / B)
        hard_ref[...] = hard_ref[...] * (1.0 / B)


@functools.partial(jax.jit)
def kernel(h, x, permutation_weights, expert_weights, bias):
    bias2d = bias.reshape(1, E)
    grid = (B // BT,)
    y, soft, hard = pl.pallas_call(
        _body,
        grid=grid,
        in_specs=[
            pl.BlockSpec((BT, D), lambda i: (i, 0)),
            pl.BlockSpec((E, D), lambda i: (0, 0)),
            pl.BlockSpec((1, E), lambda i: (0, 0)),
            pl.BlockSpec((P, E, E), lambda i: (0, 0, 0)),
            pl.BlockSpec((E, BT, D), lambda i: (0, i, 0)),
        ],
        out_specs=[
            pl.BlockSpec((BT, D), lambda i: (i, 0)),
            pl.BlockSpec((1, E), lambda i: (0, 0)),
            pl.BlockSpec((1, E), lambda i: (0, 0)),
        ],
        out_shape=[
            jax.ShapeDtypeStruct((B, D), jnp.float32),
            jax.ShapeDtypeStruct((1, E), jnp.float32),
            jax.ShapeDtypeStruct((1, E), jnp.float32),
        ],
        compiler_params=pltpu.CompilerParams(
            dimension_semantics=("arbitrary",),
        ),
    )(x, expert_weights, bias2d, permutation_weights, h)
    return (y, soft.reshape(E, 1), hard.reshape(E, 1))
